# baseline (speedup 1.0000x reference)
import functools

import jax
import jax.numpy as jnp
from jax import lax
from jax.experimental import pallas as pl
from jax.experimental.pallas import tpu as pltpu
from jax.experimental.pallas import tpu_sc as plsc


@functools.lru_cache(maxsize=None)
def _build_expand(SL: int, D: int):
    info = plsc.get_sparse_core_info()
    NC, NS = info.num_cores, info.num_subcores
    NW = NC * NS
    assert SL % NW == 0
    RPW = SL // NW
    SPAN = SL + RPW

    mesh = plsc.VectorSubcoreMesh(core_axis_name="c", subcore_axis_name="s")

    @functools.partial(
        pl.kernel,
        mesh=mesh,
        out_type=jax.ShapeDtypeStruct((SL, SL, D), jnp.float32),
        scratch_types=[
            pltpu.VMEM((SPAN, D), jnp.float32),
            pltpu.SemaphoreType.DMA,
        ],
        compiler_params=pltpu.CompilerParams(use_tc_tiling_on_sc=False),
    )
    def expand(wrev_hbm, out_hbm, stage, sem):
        wid = lax.axis_index("s") * NC + lax.axis_index("c")
        base = wid * RPW
        lo = pl.multiple_of(SL - RPW - base, 8)
        pltpu.sync_copy(wrev_hbm.at[pl.ds(lo, SPAN)], stage)
        copies = []
        for t in range(RPW):
            copies.append(
                pltpu.async_copy(
                    stage.at[pl.ds(RPW - 1 - t, SL)],
                    out_hbm.at[base + t],
                    sem,
                )
            )
        for c in copies:
            c.wait()

    return expand


def kernel(seq_len, relative_positions_weight):
    V, D = relative_positions_weight.shape
    SL = (V + 1) // 2
    wrev = jnp.flip(relative_positions_weight, axis=0)
    wrev = jnp.concatenate([wrev, jnp.zeros((1, D), wrev.dtype)], axis=0)
    return _build_expand(SL, D)(wrev)


# speedup vs baseline: 4.5637x; 4.5637x over previous
import functools

import jax
import jax.numpy as jnp
from jax import lax
from jax.experimental import pallas as pl
from jax.experimental.pallas import tpu as pltpu
from jax.experimental.pallas import tpu_sc as plsc


@functools.lru_cache(maxsize=None)
def _build_expand(SL: int, D: int):
    info = plsc.get_sparse_core_info()
    NC, NS = info.num_cores, info.num_subcores
    NW = NC * NS
    assert SL % NW == 0 and SL % 128 == 0 and D % 8 == 0
    RPW = SL // NW
    DT = D // 8
    JT = SL // 128
    HT = DT // 2
    HROWS = HT * JT * 8

    mesh = plsc.VectorSubcoreMesh(core_axis_name="c", subcore_axis_name="s")

    @functools.partial(
        pl.kernel,
        mesh=mesh,
        out_type=jax.ShapeDtypeStruct((SL * SL * D // 128, 128), jnp.float32),
        scratch_types=[
            pltpu.VMEM_SHARED((8, D, 2 * SL), jnp.float32),
            pltpu.VMEM((HROWS, 128), jnp.float32),
            pltpu.VMEM((HROWS, 128), jnp.float32),
            pltpu.SemaphoreType.DMA,
            pltpu.SemaphoreType.DMA,
            pltpu.SemaphoreType.DMA,
        ],
        compiler_params=pltpu.CompilerParams(use_tc_tiling_on_sc=False),
    )
    def expand(wtf_hbm, out_hbm, spm, buf0, buf1, gsem, ssem0, ssem1):
        cid = lax.axis_index("c")
        sid = lax.axis_index("s")
        wid = sid * NC + cid
        base = wid * RPW
        @pl.when(sid == 0)
        def _stage():
            pltpu.sync_copy(wtf_hbm, spm)
        plsc.subcore_barrier()

        bufs = (buf0, buf1)
        ssems = (ssem0, ssem1)
        PLANE_ROWS = 2 * HROWS

        def body(t, _):
            i = base + t
            c0 = SL - 1 - i
            ph = lax.rem(c0, 8)
            a0 = pl.multiple_of(c0 - ph, 8)
            for h in range(2):
                buf, ssem = bufs[h], ssems[h]
                dst_row = i * PLANE_ROWS + h * HROWS
                @pl.when(t > 0)
                def _drain():
                    pltpu.make_async_copy(
                        buf,
                        out_hbm.at[pl.ds(dst_row - PLANE_ROWS, HROWS)],
                        ssem,
                    ).wait()
                gathers = []
                for dtl in range(HT):
                    dt = h * HT + dtl
                    for jt in range(JT):
                        gathers.append(
                            pltpu.async_copy(
                                spm.at[ph, pl.ds(8 * dt, 8),
                                       pl.ds(a0 + 128 * jt, 128)],
                                buf.at[pl.ds((dtl * JT + jt) * 8, 8)],
                                gsem,
                            )
                        )
                for g in gathers:
                    g.wait()
                pltpu.async_copy(
                    buf, out_hbm.at[pl.ds(dst_row, HROWS)], ssem
                )
            return _

        lax.fori_loop(0, RPW, body, None)
        last = base + RPW - 1
        for h in range(2):
            pltpu.make_async_copy(
                bufs[h],
                out_hbm.at[pl.ds(last * PLANE_ROWS + h * HROWS, HROWS)],
                ssems[h],
            ).wait()

    return expand


def kernel(seq_len, relative_positions_weight):
    V, D = relative_positions_weight.shape
    SL = (V + 1) // 2
    wtf = jnp.flip(relative_positions_weight, axis=0).T
    wtf = jnp.concatenate([wtf, jnp.zeros((D, 9), wtf.dtype)], axis=1)
    wtf8 = jnp.stack([wtf[:, p:p + 2 * SL] for p in range(8)])
    out2d = _build_expand(SL, D)(wtf8)
    out5 = out2d.reshape(SL, D // 8, SL // 128, 8, 128)
    return out5.transpose(0, 2, 4, 1, 3).reshape(SL, SL, D)
